# baseline (speedup 1.0000x reference)
import functools

import jax
import jax.numpy as jnp
from jax import lax
from jax.experimental import pallas as pl
from jax.experimental.pallas import tpu as pltpu
from jax.experimental.pallas import tpu_sc as plsc

NC = 2
NS = 16
NW = NC * NS
NBUF = 3


def _seq_embed_call(batch, seq_len, depth):
    bpw = batch // NW
    mesh = plsc.VectorSubcoreMesh(core_axis_name="c", subcore_axis_name="s")

    @functools.partial(
        pl.kernel,
        mesh=mesh,
        out_type=jax.ShapeDtypeStruct((batch, seq_len, depth), jnp.float32),
        scratch_types=[
            pltpu.VMEM((bpw * seq_len,), jnp.int32),
            pltpu.MemorySpace.VMEM_SHARED(
                (NS, seq_len, depth), jnp.float32),
        ]
        + [pltpu.VMEM((seq_len, depth), jnp.float32) for _ in range(NBUF)]
        + [pltpu.SemaphoreType.DMA for _ in range(3 * NBUF)],
    )
    def run(seq_hbm, tok_hbm, pos_hbm, out_hbm, idx_v, pos_sh, *rest):
        bufs = rest[:NBUF]
        csems = rest[NBUF:2 * NBUF]
        gsems = rest[2 * NBUF:3 * NBUF]
        wsems = rest[3 * NBUF:]
        sub = lax.axis_index("s")
        wid = sub * NC + lax.axis_index("c")
        b0 = wid * bpw
        pltpu.sync_copy(seq_hbm.at[wid], idx_v)
        pltpu.sync_copy(pos_hbm, pos_sh.at[sub])

        def copy(s):
            return pltpu.make_async_copy(pos_sh.at[sub], bufs[s], csems[s])

        def gather(r, s):
            return pltpu.make_async_copy(
                tok_hbm.at[idx_v.at[pl.ds(r * seq_len, seq_len)]],
                bufs[s], gsems[s])

        def write(r, s):
            return pltpu.make_async_copy(bufs[s], out_hbm.at[b0 + r], wsems[s])

        def step(r, b, wait_free):
            if wait_free:
                write(r - NBUF, b).wait()
            copy(b).start()
            copy(b).wait()
            gather(r, b).start(add=True)
            gather(r - 2, (b - 2) % NBUF).wait()
            write(r - 2, (b - 2) % NBUF).start()

        for r in range(2):
            copy(r).start()
            copy(r).wait()
            gather(r, r).start(add=True)
        for r in range(2, NBUF):
            step(r, r, wait_free=False)

        def group_body(i, c):
            for db in range(NBUF):
                step(NBUF * i + db, db, wait_free=True)
            return c

        lax.fori_loop(1, bpw // NBUF, group_body, 0)

        n = bpw
        for r in range((n // NBUF) * NBUF, n):
            step(r, r % NBUF, wait_free=True)
        for r in range(n - 2, n):
            gather(r, r % NBUF).wait()
            write(r, r % NBUF).start()
        for r in range(n - NBUF, n):
            write(r, r % NBUF).wait()

    return run


def kernel(seq, token_table, pos_table):
    batch, seq_len = seq.shape
    vocab, depth = token_table.shape
    bpw = batch // NW
    assert batch % NW == 0 and bpw >= 2 * NBUF and seq_len % 8 == 0

    seq_perm = seq.reshape(NW, bpw * seq_len).astype(jnp.int32)

    return _seq_embed_call(batch, seq_len, depth)(
        seq_perm, token_table, pos_table)


# speedup vs baseline: 1.0886x; 1.0886x over previous
import functools

import jax
import jax.numpy as jnp
from jax import lax
from jax.experimental import pallas as pl
from jax.experimental.pallas import tpu as pltpu
from jax.experimental.pallas import tpu_sc as plsc

NC = 2
NS = 16
NW = NC * NS
LANES = 16
NBUF = 4


def _seq_embed_call(batch, seq_len, depth):
    bpw = batch // NW
    nvr = depth // LANES
    mesh = plsc.VectorSubcoreMesh(core_axis_name="c", subcore_axis_name="s")

    @functools.partial(
        pl.kernel,
        mesh=mesh,
        out_type=jax.ShapeDtypeStruct((batch, seq_len, depth), jnp.float32),
        scratch_types=[
            pltpu.VMEM((seq_len, bpw), jnp.int32),
            pltpu.VMEM((seq_len, depth), jnp.float32),
        ]
        + [pltpu.VMEM((bpw, depth), jnp.float32) for _ in range(NBUF)]
        + [pltpu.SemaphoreType.DMA for _ in range(2 * NBUF)],
    )
    def run(seq_hbm, tok_hbm, pos_hbm, out_hbm, idx_v, pos_v, *rest):
        bufs = rest[:NBUF]
        gsems = rest[NBUF:2 * NBUF]
        wsems = rest[2 * NBUF:]
        wid = lax.axis_index("s") * NC + lax.axis_index("c")
        b0 = wid * bpw
        pltpu.sync_copy(seq_hbm.at[wid], idx_v)
        pltpu.sync_copy(pos_hbm, pos_v)

        def gather(l, b):
            return pltpu.make_async_copy(
                tok_hbm.at[idx_v.at[l, :]], bufs[b], gsems[b])

        def write(l, b):
            return pltpu.make_async_copy(
                bufs[b], out_hbm.at[pl.ds(b0, bpw), l, :], wsems[b])

        def prefill(l, b):
            prow = [pos_v[l, pl.ds(k * LANES, LANES)] for k in range(nvr)]

            def row_body(r, c):
                for k in range(nvr):
                    bufs[b][r, pl.ds(k * LANES, LANES)] = prow[k]
                return c

            lax.fori_loop(0, bpw, row_body, 0)

        def step(l, b, wait_free):
            if wait_free:
                write(l - NBUF, b).wait()
            prefill(l, b)
            gather(l, b).start(add=True)
            gather(l - 2, (b - 2) % NBUF).wait()
            write(l - 2, (b - 2) % NBUF).start()

        for l in range(2):
            prefill(l, l)
            gather(l, l).start(add=True)
        for l in range(2, NBUF):
            step(l, l, wait_free=False)

        def group_body(i, c):
            for db in range(NBUF):
                step(NBUF * i + db, db, wait_free=True)
            return c

        lax.fori_loop(1, seq_len // NBUF, group_body, 0)

        for l in range(seq_len - 2, seq_len):
            gather(l, l % NBUF).wait()
            write(l, l % NBUF).start()
        for l in range(seq_len - NBUF, seq_len):
            write(l, l % NBUF).wait()

    return run


def kernel(seq, token_table, pos_table):
    batch, seq_len = seq.shape
    vocab, depth = token_table.shape
    bpw = batch // NW
    assert batch % NW == 0 and depth % LANES == 0 and seq_len % NBUF == 0

    seq_perm = jnp.transpose(
        seq.reshape(NW, bpw, seq_len).astype(jnp.int32), (0, 2, 1))

    return _seq_embed_call(batch, seq_len, depth)(
        seq_perm, token_table, pos_table)
